# Initial kernel scaffold; baseline (speedup 1.0000x reference)
import jax
import jax.numpy as jnp
from jax.experimental import pallas as pl


def kernel(x, edge_index, enc1_Wl, enc1_bl, enc1_Wr, enc2_Wl, enc2_bl, enc2_Wr, attr1_Wl, attr1_bl, attr1_Wr, attr2_Wl, attr2_bl, attr2_Wr, str1_Wl, str1_bl, str1_Wr, str2_Wl, str2_bl, str2_Wr):
    raise NotImplementedError("write your pallas kernel here")


# speedup vs baseline: 2.8004x; 2.8004x over previous
import functools

import jax
import jax.numpy as jnp
from jax import lax
from jax.experimental import pallas as pl
from jax.experimental.pallas import tpu as pltpu
from jax.experimental.pallas import tpu_sc as plsc

_NC = 2
_NS = 16
_NW = _NC * _NS
_CHUNK = 128


def _make_agg(n, d, e):
    kpw = e // (_NW * _CHUNK)
    rps = n // _NS

    @functools.partial(
        pl.kernel,
        out_type=jax.ShapeDtypeStruct((_NC, n, d), jnp.float32),
        mesh=plsc.VectorSubcoreMesh(core_axis_name="c", subcore_axis_name="s"),
        scratch_types=[
            pltpu.VMEM((kpw, _CHUNK), jnp.int32),
            pltpu.VMEM((kpw, _CHUNK), jnp.int32),
            pltpu.VMEM((_CHUNK, d), jnp.float32),
            pltpu.VMEM((64, d), jnp.float32),
            pltpu.VMEM_SHARED((n, d), jnp.float32),
            pltpu.SemaphoreType.DMA,
        ],
    )
    def agg(h_hbm, src_hbm, dst_hbm, out_hbm, srcv, dstv, rows, zb, accum, sem):
        c = lax.axis_index("c")
        s = lax.axis_index("s")
        w = c * _NS + s

        pltpu.sync_copy(src_hbm.at[w], srcv)
        pltpu.sync_copy(dst_hbm.at[w], dstv)

        zero = jnp.zeros((16,), jnp.float32)
        for r in range(64):
            for cc in range(d // 16):
                zb[r, pl.ds(cc * 16, 16)] = zero
        for t in range(rps // 64):
            pltpu.sync_copy(zb, accum.at[pl.ds(s * rps + t * 64, 64)])
        plsc.subcore_barrier()

        for k in range(kpw):
            pltpu.async_copy(h_hbm.at[srcv.at[k]], rows, sem).wait()
            pltpu.sync_copy(rows, accum.at[dstv.at[k]], add=True)
        plsc.subcore_barrier()

        pltpu.sync_copy(accum.at[pl.ds(s * rps, rps)],
                        out_hbm.at[c, pl.ds(s * rps, rps)])

    return agg


def _make_deg(n, e, d):
    kpw = e // (_NW * _CHUNK)
    rps = n // _NS

    @functools.partial(
        pl.kernel,
        out_type=jax.ShapeDtypeStruct((_NC, n, d), jnp.float32),
        mesh=plsc.VectorSubcoreMesh(core_axis_name="c", subcore_axis_name="s"),
        scratch_types=[
            pltpu.VMEM((kpw, _CHUNK), jnp.int32),
            pltpu.VMEM((_CHUNK, d), jnp.float32),
            pltpu.VMEM((64, d), jnp.float32),
            pltpu.VMEM_SHARED((n, d), jnp.float32),
        ],
    )
    def deg(dst_hbm, out_hbm, dstv, ones_b, zb, accum):
        c = lax.axis_index("c")
        s = lax.axis_index("s")
        w = c * _NS + s

        pltpu.sync_copy(dst_hbm.at[w], dstv)

        one = jnp.ones((16,), jnp.float32)
        zero = jnp.zeros((16,), jnp.float32)
        for r in range(_CHUNK):
            for cc in range(d // 16):
                ones_b[r, pl.ds(cc * 16, 16)] = one
        for r in range(64):
            for cc in range(d // 16):
                zb[r, pl.ds(cc * 16, 16)] = zero
        for t in range(rps // 64):
            pltpu.sync_copy(zb, accum.at[pl.ds(s * rps + t * 64, 64)])
        plsc.subcore_barrier()

        for k in range(kpw):
            pltpu.sync_copy(ones_b, accum.at[dstv.at[k]], add=True)
        plsc.subcore_barrier()

        pltpu.sync_copy(accum.at[pl.ds(s * rps, rps)],
                        out_hbm.at[c, pl.ds(s * rps, rps)])

    return deg


def _deginv_body(deg_ref, out_ref):
    d = deg_ref[0, :, 0:1] + deg_ref[1, :, 0:1]
    inv = 1.0 / jnp.maximum(d, 1.0)
    out_ref[...] = jnp.broadcast_to(inv, out_ref.shape)


def _deginv(deg, n):
    blk = 256
    return pl.pallas_call(
        _deginv_body,
        grid=(n // blk,),
        in_specs=[pl.BlockSpec((_NC, blk, 128), lambda i: (0, i, 0))],
        out_specs=pl.BlockSpec((blk, 128), lambda i: (i, 0)),
        out_shape=jax.ShapeDtypeStruct((n, 128), jnp.float32),
    )(deg)


def _dense_body(p_ref, dinv_ref, h_ref, wl_ref, bl_ref, wr_ref, out_ref):
    mean = (p_ref[0] + p_ref[1]) * dinv_ref[...]
    acc = jnp.dot(mean, wl_ref[...], preferred_element_type=jnp.float32)
    acc += jnp.dot(h_ref[...], wr_ref[...], preferred_element_type=jnp.float32)
    acc += bl_ref[...]
    out_ref[...] = jnp.maximum(acc, 0.0)


def _dense(p, dinv, h, wlT, bl, wrT, n, d_out):
    blk = 256
    d_in = h.shape[1]
    return pl.pallas_call(
        _dense_body,
        grid=(n // blk,),
        in_specs=[
            pl.BlockSpec((_NC, blk, d_in), lambda i: (0, i, 0)),
            pl.BlockSpec((blk, 128), lambda i: (i, 0)),
            pl.BlockSpec((blk, d_in), lambda i: (i, 0)),
            pl.BlockSpec((d_in, d_out), lambda i: (0, 0)),
            pl.BlockSpec((1, d_out), lambda i: (0, 0)),
            pl.BlockSpec((d_in, d_out), lambda i: (0, 0)),
        ],
        out_specs=pl.BlockSpec((blk, d_out), lambda i: (i, 0)),
        out_shape=jax.ShapeDtypeStruct((n, d_out), jnp.float32),
    )(p, dinv, h, wlT, bl.reshape(1, d_out), wrT)


def _mean_body(p_ref, dinv_ref, out_ref):
    out_ref[...] = (p_ref[0] + p_ref[1]) * dinv_ref[...]


def _meanify(p, dinv, n, d):
    blk = 256
    return pl.pallas_call(
        _mean_body,
        grid=(n // blk,),
        in_specs=[
            pl.BlockSpec((_NC, blk, d), lambda i: (0, i, 0)),
            pl.BlockSpec((blk, 128), lambda i: (i, 0)),
        ],
        out_specs=pl.BlockSpec((blk, d), lambda i: (i, 0)),
        out_shape=jax.ShapeDtypeStruct((n, d), jnp.float32),
    )(p, dinv)


def _big_body(wl_ref, bl_ref, wr_ref, mean_ref, h_ref, out_ref):
    j = pl.program_id(1)
    mean = mean_ref[pl.ds(j * 128, 128), :]
    h = h_ref[pl.ds(j * 128, 128), :]
    dn = (((1,), (1,)), ((), ()))
    acc = lax.dot_general(wl_ref[...], mean, dn,
                          preferred_element_type=jnp.float32)
    acc += lax.dot_general(wr_ref[...], h, dn,
                           preferred_element_type=jnp.float32)
    acc += bl_ref[:, 0:1]
    out_ref[...] = jnp.maximum(acc, 0.0)


def _big(wl, bl_bc, wr, mean, h, n):
    blk = 128
    return pl.pallas_call(
        _big_body,
        grid=(n // blk, n // blk),
        in_specs=[
            pl.BlockSpec((blk, 128), lambda i, j: (i, 0)),
            pl.BlockSpec((blk, 128), lambda i, j: (i, 0)),
            pl.BlockSpec((blk, 128), lambda i, j: (i, 0)),
            pl.BlockSpec((n, 128), lambda i, j: (0, 0)),
            pl.BlockSpec((n, 128), lambda i, j: (0, 0)),
        ],
        out_specs=pl.BlockSpec((blk, blk), lambda i, j: (i, j)),
        out_shape=jax.ShapeDtypeStruct((n, n), jnp.float32),
    )(wl, bl_bc, wr, mean, h)


def kernel(x, edge_index,
           enc1_Wl, enc1_bl, enc1_Wr,
           enc2_Wl, enc2_bl, enc2_Wr,
           attr1_Wl, attr1_bl, attr1_Wr,
           attr2_Wl, attr2_bl, attr2_Wr,
           str1_Wl, str1_bl, str1_Wr,
           str2_Wl, str2_bl, str2_Wr):
    n, d = x.shape
    e = edge_index.shape[1]
    kpw = e // (_NW * _CHUNK)

    e3 = edge_index.reshape(2, _NW, kpw, _CHUNK)
    src3, dst3 = e3[0], e3[1]

    agg = _make_agg(n, d, e)
    deg = _make_deg(n, e, 128)(dst3)
    dinv = _deginv(deg, n)

    def layer(h, wl, bl, wr):
        p = agg(h, src3, dst3)
        return _dense(p, dinv, h, wl.T, bl, wr.T, n, wl.shape[0])

    z = layer(x, enc1_Wl, enc1_bl, enc1_Wr)
    z = layer(z, enc2_Wl, enc2_bl, enc2_Wr)

    pz = agg(z, src3, dst3)
    a = _dense(pz, dinv, z, attr1_Wl.T, attr1_bl, attr1_Wr.T, n, 128)
    s = _dense(pz, dinv, z, str1_Wl.T, str1_bl, str1_Wr.T, n, 128)

    x_hat = layer(a, attr2_Wl, attr2_bl, attr2_Wr)

    ps = agg(s, src3, dst3)
    ms = _meanify(ps, dinv, n, d)
    bl_bc = jnp.broadcast_to(str2_bl[:, None], (n, 128))
    struct = _big(str2_Wl, bl_bc, str2_Wr, ms, s, n)

    return (struct, x_hat)


# speedup vs baseline: 6.6231x; 2.3650x over previous
import functools

import jax
import jax.numpy as jnp
from jax import lax
from jax.experimental import pallas as pl
from jax.experimental.pallas import tpu as pltpu
from jax.experimental.pallas import tpu_sc as plsc

_NC = 2
_NS = 16
_NW = _NC * _NS
_CHUNK = 128


def _make_agg(n, d, e):
    kpw = e // (_NW * _CHUNK)
    rps = n // _NS

    @functools.partial(
        pl.kernel,
        out_type=jax.ShapeDtypeStruct((_NC, n, d), jnp.float32),
        mesh=plsc.VectorSubcoreMesh(core_axis_name="c", subcore_axis_name="s"),
        scratch_types=[
            pltpu.VMEM((kpw, _CHUNK), jnp.int32),
            pltpu.VMEM((kpw, _CHUNK), jnp.int32),
            pltpu.VMEM((_CHUNK, d), jnp.float32),
            pltpu.VMEM((_CHUNK, d), jnp.float32),
            pltpu.VMEM((64, d), jnp.float32),
            pltpu.VMEM_SHARED((n, d), jnp.float32),
            pltpu.SemaphoreType.DMA,
            pltpu.SemaphoreType.DMA,
            pltpu.SemaphoreType.DMA,
        ],
    )
    def agg(h_hbm, src_hbm, dst_hbm, out_hbm,
            srcv, dstv, rows0, rows1, zb, accum, gsem, ssem0, ssem1):
        c = lax.axis_index("c")
        s = lax.axis_index("s")
        w = c * _NS + s

        pltpu.sync_copy(src_hbm.at[w], srcv)
        pltpu.sync_copy(dst_hbm.at[w], dstv)

        zero = jnp.zeros((16,), jnp.float32)
        for r in range(64):
            for cc in range(d // 16):
                zb[r, pl.ds(cc * 16, 16)] = zero
        for t in range(rps // 64):
            pltpu.sync_copy(zb, accum.at[pl.ds(s * rps + t * 64, 64)])
        plsc.subcore_barrier()

        rows = (rows0, rows1)
        ssem = (ssem0, ssem1)
        pending = [None, None]
        for k in range(kpw):
            b = k % 2
            if pending[b] is not None:
                pending[b].wait()
            pltpu.async_copy(h_hbm.at[srcv.at[k]], rows[b], gsem).wait()
            pending[b] = pltpu.async_copy(rows[b], accum.at[dstv.at[k]],
                                          ssem[b], add=True)
        for b in range(2):
            if pending[b] is not None:
                pending[b].wait()
        plsc.subcore_barrier()

        pltpu.sync_copy(accum.at[pl.ds(s * rps, rps)],
                        out_hbm.at[c, pl.ds(s * rps, rps)])

    return agg


def _make_deg(n, e, d):
    kpw = e // (_NW * _CHUNK)
    rps = n // _NS

    @functools.partial(
        pl.kernel,
        out_type=jax.ShapeDtypeStruct((_NC, n, d), jnp.float32),
        mesh=plsc.VectorSubcoreMesh(core_axis_name="c", subcore_axis_name="s"),
        scratch_types=[
            pltpu.VMEM((kpw, _CHUNK), jnp.int32),
            pltpu.VMEM((_CHUNK, d), jnp.float32),
            pltpu.VMEM((64, d), jnp.float32),
            pltpu.VMEM_SHARED((n, d), jnp.float32),
            pltpu.SemaphoreType.DMA,
            pltpu.SemaphoreType.DMA,
        ],
    )
    def deg(dst_hbm, out_hbm, dstv, ones_b, zb, accum, ssem0, ssem1):
        c = lax.axis_index("c")
        s = lax.axis_index("s")
        w = c * _NS + s

        pltpu.sync_copy(dst_hbm.at[w], dstv)

        one = jnp.ones((16,), jnp.float32)
        zero = jnp.zeros((16,), jnp.float32)
        for r in range(_CHUNK):
            for cc in range(d // 16):
                ones_b[r, pl.ds(cc * 16, 16)] = one
        for r in range(64):
            for cc in range(d // 16):
                zb[r, pl.ds(cc * 16, 16)] = zero
        for t in range(rps // 64):
            pltpu.sync_copy(zb, accum.at[pl.ds(s * rps + t * 64, 64)])
        plsc.subcore_barrier()

        ssem = (ssem0, ssem1)
        pending = [None, None]
        for k in range(kpw):
            b = k % 2
            if pending[b] is not None:
                pending[b].wait()
            pending[b] = pltpu.async_copy(ones_b, accum.at[dstv.at[k]],
                                          ssem[b], add=True)
        for b in range(2):
            if pending[b] is not None:
                pending[b].wait()
        plsc.subcore_barrier()

        pltpu.sync_copy(accum.at[pl.ds(s * rps, rps)],
                        out_hbm.at[c, pl.ds(s * rps, rps)])

    return deg


def _dense_body(p_ref, deg_ref, h_ref, wl_ref, bl_ref, wr_ref, out_ref):
    dinv = 1.0 / jnp.maximum(deg_ref[0, :, 0:1] + deg_ref[1, :, 0:1], 1.0)
    mean = (p_ref[0] + p_ref[1]) * dinv
    acc = jnp.dot(mean, wl_ref[...], preferred_element_type=jnp.float32)
    acc += jnp.dot(h_ref[...], wr_ref[...], preferred_element_type=jnp.float32)
    acc += bl_ref[...]
    out_ref[...] = jnp.maximum(acc, 0.0)


def _dense(p, deg, h, wlT, bl, wrT, n, d_out):
    blk = 1024
    d_in = h.shape[1]
    return pl.pallas_call(
        _dense_body,
        grid=(n // blk,),
        in_specs=[
            pl.BlockSpec((_NC, blk, d_in), lambda i: (0, i, 0)),
            pl.BlockSpec((_NC, blk, 128), lambda i: (0, i, 0)),
            pl.BlockSpec((blk, d_in), lambda i: (i, 0)),
            pl.BlockSpec((d_in, d_out), lambda i: (0, 0)),
            pl.BlockSpec((1, d_out), lambda i: (0, 0)),
            pl.BlockSpec((d_in, d_out), lambda i: (0, 0)),
        ],
        out_specs=pl.BlockSpec((blk, d_out), lambda i: (i, 0)),
        out_shape=jax.ShapeDtypeStruct((n, d_out), jnp.float32),
    )(p, deg, h, wlT, bl.reshape(1, d_out), wrT)


def _big_body(wl_ref, bl_ref, wr_ref, p_ref, deg_ref, h_ref, out_ref):
    blk = out_ref.shape[0]
    j = pl.program_id(1)
    rows = pl.ds(j * blk, blk)
    dinv = 1.0 / jnp.maximum(deg_ref[0, rows, 0:1] + deg_ref[1, rows, 0:1], 1.0)
    mean = (p_ref[0, rows, :] + p_ref[1, rows, :]) * dinv
    h = h_ref[rows, :]
    dn = (((1,), (1,)), ((), ()))
    acc = lax.dot_general(wl_ref[...], mean, dn,
                          preferred_element_type=jnp.float32)
    acc += lax.dot_general(wr_ref[...], h, dn,
                           preferred_element_type=jnp.float32)
    acc += bl_ref[:, 0:1]
    out_ref[...] = jnp.maximum(acc, 0.0)


def _big(wl, bl_bc, wr, p, deg, h, n):
    blk = 512
    return pl.pallas_call(
        _big_body,
        grid=(n // blk, n // blk),
        in_specs=[
            pl.BlockSpec((blk, 128), lambda i, j: (i, 0)),
            pl.BlockSpec((blk, 128), lambda i, j: (i, 0)),
            pl.BlockSpec((blk, 128), lambda i, j: (i, 0)),
            pl.BlockSpec((_NC, n, 128), lambda i, j: (0, 0, 0)),
            pl.BlockSpec((_NC, n, 128), lambda i, j: (0, 0, 0)),
            pl.BlockSpec((n, 128), lambda i, j: (0, 0)),
        ],
        out_specs=pl.BlockSpec((blk, blk), lambda i, j: (i, j)),
        out_shape=jax.ShapeDtypeStruct((n, n), jnp.float32),
    )(wl, bl_bc, wr, p, deg, h)


def kernel(x, edge_index,
           enc1_Wl, enc1_bl, enc1_Wr,
           enc2_Wl, enc2_bl, enc2_Wr,
           attr1_Wl, attr1_bl, attr1_Wr,
           attr2_Wl, attr2_bl, attr2_Wr,
           str1_Wl, str1_bl, str1_Wr,
           str2_Wl, str2_bl, str2_Wr):
    n, d = x.shape
    e = edge_index.shape[1]
    kpw = e // (_NW * _CHUNK)

    e3 = edge_index.reshape(2, _NW, kpw, _CHUNK)
    src3, dst3 = e3[0], e3[1]

    agg = _make_agg(n, d, e)
    deg = _make_deg(n, e, 128)(dst3)

    def layer(h, wl, bl, wr):
        p = agg(h, src3, dst3)
        return _dense(p, deg, h, wl.T, bl, wr.T, n, wl.shape[0])

    z = layer(x, enc1_Wl, enc1_bl, enc1_Wr)
    z = layer(z, enc2_Wl, enc2_bl, enc2_Wr)

    pz = agg(z, src3, dst3)
    a = _dense(pz, deg, z, attr1_Wl.T, attr1_bl, attr1_Wr.T, n, 128)
    s = _dense(pz, deg, z, str1_Wl.T, str1_bl, str1_Wr.T, n, 128)

    x_hat = layer(a, attr2_Wl, attr2_bl, attr2_Wr)

    ps = agg(s, src3, dst3)
    bl_bc = jnp.broadcast_to(str2_bl[:, None], (n, 128))
    struct = _big(str2_Wl, bl_bc, str2_Wr, ps, deg, s, n)

    return (struct, x_hat)
